# Initial kernel scaffold; baseline (speedup 1.0000x reference)
import functools

import jax
import jax.numpy as jnp
from jax import lax
from jax.experimental import pallas as pl
from jax.experimental.pallas import tpu as pltpu
from jax.experimental.pallas import tpu_sc as plsc

N = 10000
E = 320000
NSUB = 16
NW = 32
EPW = E // NW
CHUNK = 80
NCH = EPW // CHUNK
STRIPE = N // NSUB

_mesh = plsc.VectorSubcoreMesh(core_axis_name="c", subcore_axis_name="s")

_F32 = jnp.float32
_HI = jax.lax.Precision.HIGHEST


@functools.lru_cache(maxsize=None)
def _agg_kernel(d):
    @functools.partial(
        pl.kernel,
        mesh=_mesh,
        out_type=jax.ShapeDtypeStruct((2, N, d), _F32),
        scratch_types=[
            pltpu.VMEM((NCH, CHUNK), jnp.int32),
            pltpu.VMEM((NCH, CHUNK), jnp.int32),
            pltpu.VMEM((CHUNK, d), _F32),
            pltpu.VMEM_SHARED((N, d), _F32),
            pltpu.SemaphoreType.DMA,
        ],
    )
    def k(hs_hbm, src_hbm, dst_hbm, zeros_hbm, out_hbm,
          src_v, dst_v, rows_v, accum, sem):
        cid = lax.axis_index("c")
        sid = lax.axis_index("s")
        w = cid * NSUB + sid
        pltpu.sync_copy(src_hbm.at[w], src_v)
        pltpu.sync_copy(dst_hbm.at[w], dst_v)
        pltpu.sync_copy(zeros_hbm.at[pl.ds(sid * STRIPE, STRIPE)],
                        accum.at[pl.ds(sid * STRIPE, STRIPE)])
        plsc.subcore_barrier()

        @pl.loop(0, NCH)
        def _(j):
            pltpu.async_copy(hs_hbm.at[src_v.at[j]], rows_v, sem).wait()
            pltpu.sync_copy(rows_v, accum.at[dst_v.at[j]], add=True)

        plsc.subcore_barrier()
        pltpu.sync_copy(accum.at[pl.ds(sid * STRIPE, STRIPE)],
                        out_hbm.at[cid, pl.ds(sid * STRIPE, STRIPE)])

    return k


@functools.partial(
    pl.kernel,
    mesh=_mesh,
    out_type=jax.ShapeDtypeStruct((2, N, 16), _F32),
    scratch_types=[
        pltpu.VMEM((NCH, CHUNK), jnp.int32),
        pltpu.VMEM((CHUNK, 16), _F32),
        pltpu.VMEM_SHARED((N, 16), _F32),
        pltpu.SemaphoreType.DMA,
    ],
)
def _deg_kernel(dst_hbm, ones_hbm, zeros_hbm, out_hbm,
                dst_v, ones_v, accum, sem):
    cid = lax.axis_index("c")
    sid = lax.axis_index("s")
    w = cid * NSUB + sid
    pltpu.sync_copy(dst_hbm.at[w], dst_v)
    pltpu.sync_copy(ones_hbm, ones_v)
    pltpu.sync_copy(zeros_hbm.at[pl.ds(sid * STRIPE, STRIPE)],
                    accum.at[pl.ds(sid * STRIPE, STRIPE)])
    plsc.subcore_barrier()

    @pl.loop(0, NCH)
    def _(j):
        pltpu.sync_copy(ones_v, accum.at[dst_v.at[j]], add=True)

    plsc.subcore_barrier()
    pltpu.sync_copy(accum.at[pl.ds(sid * STRIPE, STRIPE)],
                    out_hbm.at[cid, pl.ds(sid * STRIPE, STRIPE)])


def _dinv_body(dp_ref, o_ref):
    deg = dp_ref[0, :, 0:1] + dp_ref[1, :, 0:1] + 1.0
    o_ref[...] = lax.rsqrt(deg)


def _dinv(deg_parts):
    return pl.pallas_call(
        _dinv_body,
        out_shape=jax.ShapeDtypeStruct((N, 1), _F32),
    )(deg_parts)


def _matmul_scale_body(x_ref, w_ref, dinv_ref, o_ref):
    h = jnp.dot(x_ref[...], w_ref[...], preferred_element_type=_F32,
                precision=_HI)
    o_ref[...] = h * dinv_ref[...]


def _matmul_scale(x, W, dinv, rows=2000):
    din, dout = W.shape
    return pl.pallas_call(
        _matmul_scale_body,
        grid=(N // rows,),
        in_specs=[
            pl.BlockSpec((rows, din), lambda i: (i, 0)),
            pl.BlockSpec((din, dout), lambda i: (0, 0)),
            pl.BlockSpec((rows, 1), lambda i: (i, 0)),
        ],
        out_specs=pl.BlockSpec((rows, dout), lambda i: (i, 0)),
        out_shape=jax.ShapeDtypeStruct((N, dout), _F32),
    )(x, W, dinv)


def _combine_matmul_body(p_ref, hs_ref, dinv_ref, w_ref, b_ref, o_ref):
    a = (p_ref[0] + p_ref[1] + hs_ref[...]) * dinv_ref[...]
    g = jnp.dot(a, w_ref[...], preferred_element_type=_F32, precision=_HI)
    o_ref[...] = g + b_ref[...]


def _combine_matmul(parts, hs, dinv, W, b, rows=2000):
    din, dout = W.shape
    return pl.pallas_call(
        _combine_matmul_body,
        grid=(N // rows,),
        in_specs=[
            pl.BlockSpec((2, rows, din), lambda i: (0, i, 0)),
            pl.BlockSpec((rows, din), lambda i: (i, 0)),
            pl.BlockSpec((rows, 1), lambda i: (i, 0)),
            pl.BlockSpec((din, dout), lambda i: (0, 0)),
            pl.BlockSpec((1, dout), lambda i: (0, 0)),
        ],
        out_specs=pl.BlockSpec((rows, dout), lambda i: (i, 0)),
        out_shape=jax.ShapeDtypeStruct((N, dout), _F32),
    )(parts, hs, dinv, W, b.reshape(1, dout))


def _combine_bias_body(p_ref, hs_ref, dinv_ref, b_ref, o_ref):
    o_ref[...] = (p_ref[0] + p_ref[1] + hs_ref[...]) * dinv_ref[...] \
        + b_ref[...]


def _combine_bias(parts, hs, dinv, b, rows=2000):
    d = hs.shape[1]
    return pl.pallas_call(
        _combine_bias_body,
        grid=(N // rows,),
        in_specs=[
            pl.BlockSpec((2, rows, d), lambda i: (0, i, 0)),
            pl.BlockSpec((rows, d), lambda i: (i, 0)),
            pl.BlockSpec((rows, 1), lambda i: (i, 0)),
            pl.BlockSpec((1, d), lambda i: (0, 0)),
        ],
        out_specs=pl.BlockSpec((rows, d), lambda i: (i, 0)),
        out_shape=jax.ShapeDtypeStruct((N, d), _F32),
    )(parts, hs, dinv, b.reshape(1, d))


def _bn_relu_body(g_ref, gamma_ref, beta_ref, dinv_ref, o_ref, *, scale):
    g = g_ref[...]
    mean = jnp.mean(g, axis=0, keepdims=True)
    var = jnp.mean((g - mean) ** 2, axis=0, keepdims=True)
    z = (g - mean) * lax.rsqrt(var + 1e-5) * gamma_ref[...] + beta_ref[...]
    h = jnp.maximum(z, 0.0)
    if scale:
        h = h * dinv_ref[...]
    o_ref[...] = h


def _bn_relu(g, gamma, beta, dinv, scale):
    d = g.shape[1]
    cb = min(d, 128)
    return pl.pallas_call(
        functools.partial(_bn_relu_body, scale=scale),
        grid=(d // cb,),
        in_specs=[
            pl.BlockSpec((N, cb), lambda i: (0, i)),
            pl.BlockSpec((1, cb), lambda i: (0, i)),
            pl.BlockSpec((1, cb), lambda i: (0, i)),
            pl.BlockSpec((N, 1), lambda i: (0, 0)),
        ],
        out_specs=pl.BlockSpec((N, cb), lambda i: (0, i)),
        out_shape=jax.ShapeDtypeStruct((N, d), _F32),
    )(g, gamma.reshape(1, d), beta.reshape(1, d), dinv)


def _final_body(p_ref, hs_ref, dinv_ref, b_ref, o_ref):
    g = (p_ref[0] + p_ref[1] + hs_ref[...]) * dinv_ref[...] + b_ref[...]
    z = g[:, :40]
    m = jnp.max(z, axis=1, keepdims=True)
    zs = z - m
    o_ref[...] = zs - jnp.log(jnp.sum(jnp.exp(zs), axis=1, keepdims=True))


def _final(parts, hs, dinv, b_pad, rows=2000):
    return pl.pallas_call(
        _final_body,
        grid=(N // rows,),
        in_specs=[
            pl.BlockSpec((2, rows, 48), lambda i: (0, i, 0)),
            pl.BlockSpec((rows, 48), lambda i: (i, 0)),
            pl.BlockSpec((rows, 1), lambda i: (i, 0)),
            pl.BlockSpec((1, 48), lambda i: (0, 0)),
        ],
        out_specs=pl.BlockSpec((rows, 40), lambda i: (i, 0)),
        out_shape=jax.ShapeDtypeStruct((N, 40), _F32),
    )(parts, hs, dinv, b_pad.reshape(1, 48))


def kernel(x, edge_index, Ws, bs, gammas, betas):
    ei = edge_index.astype(jnp.int32)
    src3 = ei[0].reshape(NW, NCH, CHUNK)
    dst3 = ei[1].reshape(NW, NCH, CHUNK)

    zeros = {d: jnp.zeros((N, d), _F32) for d in (16, 32, 48, 64, 128)}
    ones16 = jnp.ones((CHUNK, 16), _F32)

    def agg(hs):
        d = hs.shape[1]
        if d <= 128:
            return _agg_kernel(d)(hs, src3, dst3, zeros[d])
        chunks = [
            _agg_kernel(128)(hs[:, c:c + 128], src3, dst3, zeros[128])
            for c in range(0, d, 128)
        ]
        return jnp.concatenate(chunks, axis=2)

    deg_parts = _deg_kernel(dst3, ones16, zeros[16])
    dinv = _dinv(deg_parts)

    hs = _matmul_scale(x, Ws[0], dinv)
    parts = agg(hs)
    g = _combine_bias(parts, hs, dinv, bs[0])
    h = _bn_relu(g, gammas[0], betas[0], dinv, scale=True)

    for i in range(1, 6):
        parts = agg(h)
        g = _combine_matmul(parts, h, dinv, Ws[i], bs[i])
        h = _bn_relu(g, gammas[i], betas[i], dinv, scale=(i < 5))

    W7 = jnp.pad(Ws[6], ((0, 0), (0, 8)))
    b7 = jnp.pad(bs[6], (0, 8))
    hs7 = _matmul_scale(h, W7, dinv)
    parts = agg(hs7)
    return _final(parts, hs7, dinv, b7)


# speedup vs baseline: 12.7354x; 12.7354x over previous
import functools

import jax
import jax.numpy as jnp
from jax import lax
from jax.experimental import pallas as pl
from jax.experimental.pallas import tpu as pltpu
from jax.experimental.pallas import tpu_sc as plsc

N = 10000
E = 320000
NSUB = 16
NW = 32
EPW = E // NW
CHUNK = 80
NCH = EPW // CHUNK
PADN = 10240
STRIPE = PADN // NSUB

_mesh = plsc.VectorSubcoreMesh(core_axis_name="c", subcore_axis_name="s")

_F32 = jnp.float32
_HI = jax.lax.Precision.HIGHEST


@functools.partial(
    pl.kernel,
    mesh=_mesh,
    out_type=jax.ShapeDtypeStruct((2, PADN, 128), _F32),
    scratch_types=[
        pltpu.VMEM((NCH, CHUNK), jnp.int32),
        pltpu.VMEM((NCH, CHUNK), jnp.int32),
        pltpu.VMEM((CHUNK, 128), _F32),
        pltpu.VMEM_SHARED((PADN, 128), _F32),
        pltpu.SemaphoreType.DMA,
    ],
)
def _agg_kernel(hs_hbm, src_hbm, dst_hbm, zeros_hbm, out_hbm,
                src_v, dst_v, rows_v, accum, sem):
    cid = lax.axis_index("c")
    sid = lax.axis_index("s")
    w = cid * NSUB + sid
    pltpu.sync_copy(src_hbm.at[w], src_v)
    pltpu.sync_copy(dst_hbm.at[w], dst_v)
    pltpu.sync_copy(zeros_hbm.at[pl.ds(sid * STRIPE, STRIPE)],
                    accum.at[pl.ds(sid * STRIPE, STRIPE)])
    plsc.subcore_barrier()

    @pl.loop(0, NCH)
    def _(j):
        pltpu.async_copy(hs_hbm.at[src_v.at[j]], rows_v, sem).wait()
        pltpu.sync_copy(rows_v, accum.at[dst_v.at[j]], add=True)

    plsc.subcore_barrier()
    pltpu.sync_copy(accum.at[pl.ds(sid * STRIPE, STRIPE)],
                    out_hbm.at[cid, pl.ds(sid * STRIPE, STRIPE)])


@functools.partial(
    pl.kernel,
    mesh=_mesh,
    out_type=jax.ShapeDtypeStruct((2, PADN, 16), _F32),
    scratch_types=[
        pltpu.VMEM((NCH, CHUNK), jnp.int32),
        pltpu.VMEM((CHUNK, 16), _F32),
        pltpu.VMEM_SHARED((PADN, 16), _F32),
    ],
)
def _deg_kernel(dst_hbm, ones_hbm, zeros_hbm, out_hbm,
                dst_v, ones_v, accum):
    cid = lax.axis_index("c")
    sid = lax.axis_index("s")
    w = cid * NSUB + sid
    pltpu.sync_copy(dst_hbm.at[w], dst_v)
    pltpu.sync_copy(ones_hbm, ones_v)
    pltpu.sync_copy(zeros_hbm.at[pl.ds(sid * STRIPE, STRIPE)],
                    accum.at[pl.ds(sid * STRIPE, STRIPE)])
    plsc.subcore_barrier()

    @pl.loop(0, NCH)
    def _(j):
        pltpu.sync_copy(ones_v, accum.at[dst_v.at[j]], add=True)

    plsc.subcore_barrier()
    pltpu.sync_copy(accum.at[pl.ds(sid * STRIPE, STRIPE)],
                    out_hbm.at[cid, pl.ds(sid * STRIPE, STRIPE)])


def _dinv_body(dp_ref, o_ref):
    deg = dp_ref[0, :, 0:1] + dp_ref[1, :, 0:1] + 1.0
    o_ref[...] = lax.rsqrt(deg)


def _dinv(deg_parts):
    return pl.pallas_call(
        _dinv_body,
        grid=(1,),
        in_specs=[pl.BlockSpec((2, N, 16), lambda i: (0, 0, 0))],
        out_specs=pl.BlockSpec((N, 1), lambda i: (0, 0)),
        out_shape=jax.ShapeDtypeStruct((N, 1), _F32),
    )(deg_parts)


def _matmul_scale_body(x_ref, w_ref, dinv_ref, o_ref):
    h = jnp.dot(x_ref[...], w_ref[...], preferred_element_type=_F32,
                precision=_HI)
    o_ref[...] = h * dinv_ref[...]


def _matmul_scale(x, W, dinv, rows=2000):
    din, dout = W.shape
    return pl.pallas_call(
        _matmul_scale_body,
        grid=(N // rows,),
        in_specs=[
            pl.BlockSpec((rows, din), lambda i: (i, 0)),
            pl.BlockSpec((din, dout), lambda i: (0, 0)),
            pl.BlockSpec((rows, 1), lambda i: (i, 0)),
        ],
        out_specs=pl.BlockSpec((rows, dout), lambda i: (i, 0)),
        out_shape=jax.ShapeDtypeStruct((N, dout), _F32),
    )(x, W, dinv)


def _combine_matmul_body(p_ref, hs_ref, dinv_ref, w_ref, b_ref, o_ref, *,
                         din):
    a = (p_ref[0][:, :din] + p_ref[1][:, :din] + hs_ref[...]) * dinv_ref[...]
    g = jnp.dot(a, w_ref[...], preferred_element_type=_F32, precision=_HI)
    o_ref[...] = g + b_ref[...]


def _combine_matmul(parts, hs, dinv, W, b, rows=2000):
    din, dout = W.shape
    return pl.pallas_call(
        functools.partial(_combine_matmul_body, din=din),
        grid=(N // rows,),
        in_specs=[
            pl.BlockSpec((2, rows, 128), lambda i: (0, i, 0)),
            pl.BlockSpec((rows, din), lambda i: (i, 0)),
            pl.BlockSpec((rows, 1), lambda i: (i, 0)),
            pl.BlockSpec((din, dout), lambda i: (0, 0)),
            pl.BlockSpec((1, dout), lambda i: (0, 0)),
        ],
        out_specs=pl.BlockSpec((rows, dout), lambda i: (i, 0)),
        out_shape=jax.ShapeDtypeStruct((N, dout), _F32),
    )(parts, hs, dinv, W, b.reshape(1, dout))


def _combine_matmul2_body(pa_ref, pb_ref, hs_ref, dinv_ref, w_ref, b_ref,
                          o_ref):
    a = jnp.concatenate(
        [pa_ref[0] + pa_ref[1], pb_ref[0] + pb_ref[1]], axis=1)
    a = (a + hs_ref[...]) * dinv_ref[...]
    g = jnp.dot(a, w_ref[...], preferred_element_type=_F32, precision=_HI)
    o_ref[...] = g + b_ref[...]


def _combine_matmul2(parts_a, parts_b, hs, dinv, W, b, rows=1000):
    din, dout = W.shape
    return pl.pallas_call(
        _combine_matmul2_body,
        grid=(N // rows,),
        in_specs=[
            pl.BlockSpec((2, rows, 128), lambda i: (0, i, 0)),
            pl.BlockSpec((2, rows, 128), lambda i: (0, i, 0)),
            pl.BlockSpec((rows, din), lambda i: (i, 0)),
            pl.BlockSpec((rows, 1), lambda i: (i, 0)),
            pl.BlockSpec((din, dout), lambda i: (0, 0)),
            pl.BlockSpec((1, dout), lambda i: (0, 0)),
        ],
        out_specs=pl.BlockSpec((rows, dout), lambda i: (i, 0)),
        out_shape=jax.ShapeDtypeStruct((N, dout), _F32),
    )(parts_a, parts_b, hs, dinv, W, b.reshape(1, dout))


def _combine_bias_body(p_ref, hs_ref, dinv_ref, b_ref, o_ref, *, d):
    o_ref[...] = (p_ref[0][:, :d] + p_ref[1][:, :d] + hs_ref[...]) \
        * dinv_ref[...] + b_ref[...]


def _combine_bias(parts, hs, dinv, b, rows=2000):
    d = hs.shape[1]
    return pl.pallas_call(
        functools.partial(_combine_bias_body, d=d),
        grid=(N // rows,),
        in_specs=[
            pl.BlockSpec((2, rows, 128), lambda i: (0, i, 0)),
            pl.BlockSpec((rows, d), lambda i: (i, 0)),
            pl.BlockSpec((rows, 1), lambda i: (i, 0)),
            pl.BlockSpec((1, d), lambda i: (0, 0)),
        ],
        out_specs=pl.BlockSpec((rows, d), lambda i: (i, 0)),
        out_shape=jax.ShapeDtypeStruct((N, d), _F32),
    )(parts, hs, dinv, b.reshape(1, d))


def _bn_relu_body(g_ref, gamma_ref, beta_ref, dinv_ref, o_ref, *, scale):
    g = g_ref[...]
    mean = jnp.mean(g, axis=0, keepdims=True)
    var = jnp.mean((g - mean) ** 2, axis=0, keepdims=True)
    z = (g - mean) * lax.rsqrt(var + 1e-5) * gamma_ref[...] + beta_ref[...]
    h = jnp.maximum(z, 0.0)
    if scale:
        h = h * dinv_ref[...]
    o_ref[...] = h


def _bn_relu(g, gamma, beta, dinv, scale):
    d = g.shape[1]
    cb = min(d, 128)
    return pl.pallas_call(
        functools.partial(_bn_relu_body, scale=scale),
        grid=(d // cb,),
        in_specs=[
            pl.BlockSpec((N, cb), lambda i: (0, i)),
            pl.BlockSpec((1, cb), lambda i: (0, i)),
            pl.BlockSpec((1, cb), lambda i: (0, i)),
            pl.BlockSpec((N, 1), lambda i: (0, 0)),
        ],
        out_specs=pl.BlockSpec((N, cb), lambda i: (0, i)),
        out_shape=jax.ShapeDtypeStruct((N, d), _F32),
    )(g, gamma.reshape(1, d), beta.reshape(1, d), dinv)


def _final_body(p_ref, hs_ref, dinv_ref, b_ref, o_ref):
    g = (p_ref[0][:, :48] + p_ref[1][:, :48] + hs_ref[...]) * dinv_ref[...] \
        + b_ref[...]
    z = g[:, :40]
    m = jnp.max(z, axis=1, keepdims=True)
    zs = z - m
    o_ref[...] = zs - jnp.log(jnp.sum(jnp.exp(zs), axis=1, keepdims=True))


def _final(parts, hs, dinv, b_pad, rows=2000):
    return pl.pallas_call(
        _final_body,
        grid=(N // rows,),
        in_specs=[
            pl.BlockSpec((2, rows, 128), lambda i: (0, i, 0)),
            pl.BlockSpec((rows, 48), lambda i: (i, 0)),
            pl.BlockSpec((rows, 1), lambda i: (i, 0)),
            pl.BlockSpec((1, 48), lambda i: (0, 0)),
        ],
        out_specs=pl.BlockSpec((rows, 40), lambda i: (i, 0)),
        out_shape=jax.ShapeDtypeStruct((N, 40), _F32),
    )(parts, hs, dinv, b_pad.reshape(1, 48))


def kernel(x, edge_index, Ws, bs, gammas, betas):
    ei = edge_index.astype(jnp.int32)
    src3 = ei[0].reshape(NW, NCH, CHUNK)
    dst3 = ei[1].reshape(NW, NCH, CHUNK)

    zeros128 = jnp.zeros((PADN, 128), _F32)
    zeros16 = jnp.zeros((PADN, 16), _F32)
    ones16 = jnp.ones((CHUNK, 16), _F32)

    def agg(hs):
        d = hs.shape[1]
        if d < 128:
            hs = jnp.pad(hs, ((0, 0), (0, 128 - d)))
        return _agg_kernel(hs, src3, dst3, zeros128)

    deg_parts = _deg_kernel(dst3, ones16, zeros16)
    dinv = _dinv(deg_parts)

    hs = _matmul_scale(x, Ws[0], dinv)
    parts = agg(hs)
    g = _combine_bias(parts, hs, dinv, bs[0])
    h = _bn_relu(g, gammas[0], betas[0], dinv, scale=True)

    for i in range(1, 5):
        parts = agg(h)
        g = _combine_matmul(parts, h, dinv, Ws[i], bs[i])
        h = _bn_relu(g, gammas[i], betas[i], dinv, scale=True)

    parts_a = agg(lax.slice(h, (0, 0), (N, 128)))
    parts_b = agg(lax.slice(h, (0, 128), (N, 256)))
    g = _combine_matmul2(parts_a, parts_b, h, dinv, Ws[5], bs[5])
    h = _bn_relu(g, gammas[5], betas[5], dinv, scale=False)

    W7 = jnp.pad(Ws[6], ((0, 0), (0, 8)))
    b7 = jnp.pad(bs[6], (0, 8))
    hs7 = _matmul_scale(h, W7, dinv)
    parts = agg(hs7)
    return _final(parts, hs7, dinv, b7)
